# baseline (speedup 1.0000x reference)
import functools

import jax
import jax.numpy as jnp
from jax import lax
from jax.experimental import pallas as pl
from jax.experimental.pallas import tpu as pltpu
from jax.experimental.pallas import tpu_sc as plsc

jax.config.update("jax_enable_x64", True)

_NC = 2
_NS = 16
_NW = _NC * _NS
_LANES = 16
_ROWS = 11 * 64
_RPW = 24
_PADROWS = _NW * _RPW
_WORDS = 128 // 4

_mesh = plsc.VectorSubcoreMesh(core_axis_name="c", subcore_axis_name="s")


@functools.partial(
    pl.kernel,
    mesh=_mesh,
    out_type=[
        jax.ShapeDtypeStruct((_PADROWS, _WORDS), jnp.int32),
        jax.ShapeDtypeStruct((11, 128), jnp.float32),
    ],
    scratch_types=[
        pltpu.VMEM((_RPW, _LANES), jnp.int32),
        pltpu.VMEM((_RPW, _WORDS), jnp.int32),
        pltpu.VMEM((16,), jnp.int32),
        pltpu.VMEM((16, 128), jnp.float32),
        pltpu.SemaphoreType.DMA,
    ],
)
def _sc_mask_gather(thr_hbm, idx_hbm, table_hbm, mask_out, rows_out,
                    thr_v, mask_v, idx_v, rows_v, sem):
    wid = lax.axis_index("s") * _NC + lax.axis_index("c")
    base = wid * _RPW

    @pl.when(wid == 0)
    def _gather_start():
        pltpu.sync_copy(idx_hbm, idx_v)
        pltpu.make_async_copy(table_hbm.at[idx_v], rows_v, sem).start()

    pltpu.sync_copy(thr_hbm.at[pl.ds(base, _RPW)], thr_v)
    word0 = lax.iota(jnp.int32, _LANES)

    def body(r, carry):
        tvec = thr_v[r]
        for k in range(_WORDS // _LANES):
            d = tvec - 4 * (word0 + (k * _LANES))
            w = jnp.where(
                d >= 4, jnp.int32(0x01010101),
                jnp.where(d >= 3, jnp.int32(0x010101),
                          jnp.where(d >= 2, jnp.int32(0x0101),
                                    jnp.where(d >= 1, jnp.int32(1),
                                              jnp.int32(0)))))
            mask_v[r, pl.ds(k * _LANES, _LANES)] = w
        return carry

    lax.fori_loop(0, _RPW, body, 0)
    pltpu.sync_copy(mask_v, mask_out.at[pl.ds(base, _RPW)])

    @pl.when(wid == 0)
    def _gather_finish():
        pltpu.make_async_copy(table_hbm.at[idx_v], rows_v, sem).wait()
        pltpu.sync_copy(rows_v.at[pl.ds(0, 11)], rows_out)


def kernel(primals_1, primals_2, primals_3, primals_4):
    p2 = primals_2.astype(jnp.int32)
    ct1 = primals_3.astype(jnp.int32)
    thr1d = jnp.pad(p2[:, :, 0].reshape(-1), (0, _PADROWS - _ROWS))
    thr = jnp.broadcast_to(thr1d[:, None], (_PADROWS, _LANES))
    select_2 = p2[:, 0, 2]
    idx16 = jnp.pad(select_2, (0, 16 - select_2.shape[0]))
    mask_words, index = _sc_mask_gather(thr, idx16, primals_4)
    mask_bytes = lax.bitcast_convert_type(mask_words[:_ROWS], jnp.uint8)
    lt = mask_bytes.reshape(_ROWS, 128)[:, :120].astype(jnp.bool_)
    lt = lt.reshape(11, 64, 120)
    z0 = jnp.zeros((11, 6, 128), jnp.float64)
    z1 = jnp.zeros((11, 32, 128), jnp.float64)
    z2 = jnp.zeros((11, 128), jnp.float64)
    return (primals_1, ct1, z0, z1, z2, lt, index, select_2)


# speedup vs baseline: 1.0556x; 1.0556x over previous
import functools

import jax
import jax.numpy as jnp
from jax import lax
from jax.experimental import pallas as pl
from jax.experimental.pallas import tpu as pltpu
from jax.experimental.pallas import tpu_sc as plsc

jax.config.update("jax_enable_x64", True)

_NC = 2
_NS = 16
_NW = _NC * _NS
_LANES = 16
_ROWS = 11 * 64
_RPW = 24
_PADROWS = _NW * _RPW

_mesh = plsc.VectorSubcoreMesh(core_axis_name="c", subcore_axis_name="s")


@functools.partial(
    pl.kernel,
    mesh=_mesh,
    out_type=[
        jax.ShapeDtypeStruct((_PADROWS, 128), jnp.int32),
        jax.ShapeDtypeStruct((11, 128), jnp.float32),
        jax.ShapeDtypeStruct((11,), jnp.int32),
    ],
    scratch_types=[
        pltpu.VMEM((_RPW, _LANES), jnp.int32),
        pltpu.VMEM((_RPW, 128), jnp.int32),
        pltpu.VMEM((1, 16), jnp.int32),
        pltpu.VMEM((16, 128), jnp.float32),
        pltpu.SemaphoreType.DMA,
    ],
)
def _sc_mask_gather(prep_hbm, table_hbm, mask_out, rows_out, sel_out,
                    thr_v, mask_v, idx_v, rows_v, sem):
    wid = lax.axis_index("s") * _NC + lax.axis_index("c")
    base = wid * _RPW

    @pl.when(wid == 0)
    def _gather_start():
        pltpu.sync_copy(prep_hbm.at[pl.ds(jnp.int32(_PADROWS), 1)], idx_v)
        pltpu.make_async_copy(
            table_hbm.at[idx_v.at[jnp.int32(0)]], rows_v, sem).start()

    pltpu.sync_copy(prep_hbm.at[pl.ds(base, _RPW)], thr_v)
    col0 = lax.iota(jnp.int32, _LANES)

    for r in range(_RPW):
        tvec = thr_v[r]
        for k in range(128 // _LANES):
            col = col0 + (k * _LANES)
            val = jnp.where(col < tvec, jnp.int32(1), jnp.int32(0))
            mask_v[r, pl.ds(k * _LANES, _LANES)] = val

    pltpu.sync_copy(mask_v, mask_out.at[pl.ds(base, _RPW)])

    @pl.when(wid == 0)
    def _gather_finish():
        pltpu.make_async_copy(
            table_hbm.at[idx_v.at[jnp.int32(0)]], rows_v, sem).wait()
        pltpu.sync_copy(rows_v.at[pl.ds(0, 11)], rows_out)
        pltpu.sync_copy(idx_v.at[jnp.int32(0), pl.ds(0, 11)], sel_out)


def kernel(primals_1, primals_2, primals_3, primals_4):
    p2 = primals_2.astype(jnp.int32)
    ct1 = primals_3.astype(jnp.int32)
    thr1d = jnp.pad(p2[:, :, 0].reshape(-1), (0, _PADROWS - _ROWS))
    thr = jnp.broadcast_to(thr1d[:, None], (_PADROWS, _LANES))
    idx16 = jnp.pad(p2[:, 0, 2], (0, 16 - 11))
    prep = jnp.concatenate([thr, idx16[None, :]], axis=0)
    mask_i32, index, select_2 = _sc_mask_gather(prep, primals_4)
    lt = mask_i32[:_ROWS, :120].astype(jnp.bool_).reshape(11, 64, 120)
    z0 = jnp.zeros((11, 6, 128), jnp.float64)
    z1 = jnp.zeros((11, 32, 128), jnp.float64)
    z2 = jnp.zeros((11, 128), jnp.float64)
    return (primals_1, ct1, z0, z1, z2, lt, index, select_2)
